# baseline (speedup 1.0000x reference)
import jax
import jax.numpy as jnp
from jax.experimental import pallas as pl
from jax.experimental.pallas import tpu as pltpu

_NUM_CHANNELS = 5

_GRID = 25


def _body(x_ref, e_ref, u_ref, xo_ref, eo_ref, uo_ref):
    scale = jnp.float32(1.0 / _NUM_CHANNELS)

    def channel_mean(v):
        acc = v
        for _ in range(_NUM_CHANNELS - 1):
            acc = acc + v
        return acc * scale

    xo_ref[...] = channel_mean(x_ref[...])
    eo_ref[...] = channel_mean(e_ref[...])
    uo_ref[...] = channel_mean(u_ref[...])


def kernel(x, edge_index, edge_attr, u, batch):
    n, d = x.shape
    e, de = edge_attr.shape
    xb = n // _GRID
    eb = e // _GRID

    x_out, e_out, u_out = pl.pallas_call(
        _body,
        grid=(_GRID,),
        in_specs=[
            pl.BlockSpec((xb, d), lambda i: (i, 0)),
            pl.BlockSpec((eb, de), lambda i: (i, 0)),
            pl.BlockSpec((1, d), lambda i: (0, 0)),
        ],
        out_specs=[
            pl.BlockSpec((xb, d), lambda i: (i, 0)),
            pl.BlockSpec((eb, de), lambda i: (i, 0)),
            pl.BlockSpec((1, d), lambda i: (0, 0)),
        ],
        out_shape=[
            jax.ShapeDtypeStruct((n, d), x.dtype),
            jax.ShapeDtypeStruct((e, de), edge_attr.dtype),
            jax.ShapeDtypeStruct((1, d), u.dtype),
        ],
        compiler_params=pltpu.CompilerParams(
            dimension_semantics=("arbitrary",),
        ),
    )(x, edge_attr, u)

    return (x_out[:, :, None], e_out[:, :, None], u_out[:, :, None])
